# baseline (speedup 1.0000x reference)
import functools

import jax
import jax.numpy as jnp
import numpy as np
from jax.experimental import pallas as pl


def _mlp_kernel(nlayers, pre_relu, relu_flags, reduce_rows, *refs):
    x_ref = refs[0]
    out_ref = refs[-1]
    h = x_ref[...]
    if pre_relu:
        h = jnp.maximum(h, 0.0)
    for i in range(nlayers):
        W = refs[1 + 2 * i][...]
        b = refs[2 + 2 * i][...]
        h = jnp.dot(h, W, preferred_element_type=jnp.float32) + b
        if relu_flags[i]:
            h = jnp.maximum(h, 0.0)
    if reduce_rows is None:
        out_ref[...] = h
    else:
        total_rows, block_rows = reduce_rows
        i = pl.program_id(0)
        row = i * block_rows + jax.lax.broadcasted_iota(
            jnp.int32, (block_rows, 1), 0)
        h = jnp.where(row < total_rows, h, -jnp.inf)
        bmax = jnp.max(h, axis=0, keepdims=True)

        @pl.when(i == 0)
        def _():
            out_ref[...] = bmax

        @pl.when(i > 0)
        def _():
            out_ref[...] = jnp.maximum(out_ref[...], bmax)


def _run_mlp(layers, x, block_rows, pre_relu=False, relu_flags=None,
             reduce_rows=False):
    R, C0 = x.shape
    nl = len(layers)
    if relu_flags is None:
        relu_flags = (True,) * nl
    Rp = -(-R // block_rows) * block_rows
    if Rp != R:
        x = jnp.pad(x, ((0, Rp - R), (0, 0)))
    grid = (Rp // block_rows,)
    Cout = layers[-1][0].shape[1]

    in_specs = [pl.BlockSpec((block_rows, C0), lambda i: (i, 0))]
    args = [x]
    for (W, b) in layers:
        in_specs.append(pl.BlockSpec(W.shape, lambda i: (0, 0)))
        in_specs.append(pl.BlockSpec((1, W.shape[1]), lambda i: (0, 0)))
        args.append(W)
        args.append(b.reshape(1, -1))

    if reduce_rows:
        out_spec = pl.BlockSpec((1, Cout), lambda i: (0, 0))
        out_shape = jax.ShapeDtypeStruct((1, Cout), jnp.float32)
        red = (R, block_rows)
    else:
        out_spec = pl.BlockSpec((block_rows, Cout), lambda i: (i, 0))
        out_shape = jax.ShapeDtypeStruct((Rp, Cout), jnp.float32)
        red = None

    kern = functools.partial(_mlp_kernel, nl, pre_relu, tuple(relu_flags), red)
    out = pl.pallas_call(
        kern, grid=grid, in_specs=in_specs, out_specs=out_spec,
        out_shape=out_shape)(*args)
    if not reduce_rows and Rp != R:
        out = out[:R]
    return out


def _fps_kernel(n, m, Mp, pos_ref, out_ref):
    pos = pos_ref[...]
    Np = pos.shape[1]
    lane = jax.lax.broadcasted_iota(jnp.int32, (1, Np), 1)
    valid = lane < n
    lane_m = jax.lax.broadcasted_iota(jnp.int32, (1, Mp), 1)
    out_ref[...] = jnp.zeros((1, Mp), jnp.int32)

    def body(i, carry):
        dmin, prev = carry
        cur = jnp.sum(jnp.where(lane == prev, pos, 0.0), axis=1,
                      keepdims=True)
        d = jnp.sum((pos - cur) ** 2, axis=0, keepdims=True)
        d = jnp.where(valid, d, -jnp.inf)
        dmin = jnp.minimum(dmin, d)
        mx = jnp.max(dmin)
        j = jnp.min(jnp.where(dmin == mx, lane, Np)).astype(jnp.int32)
        out_ref[...] = jnp.where(lane_m == i, j, out_ref[...])
        return dmin, j

    dmin0 = jnp.where(valid, jnp.inf, -jnp.inf)
    jax.lax.fori_loop(1, m, body, (dmin0, jnp.int32(0)))


def _fps(pos, m):
    n = pos.shape[0]
    Np = -(-n // 128) * 128
    Mp = -(-m // 128) * 128
    pos_t = jnp.zeros((8, Np), jnp.float32).at[:3, :n].set(pos.T)
    out = pl.pallas_call(
        functools.partial(_fps_kernel, n, m, Mp),
        out_shape=jax.ShapeDtypeStruct((1, Mp), jnp.int32),
    )(pos_t)
    return out[0, :m]


def _point_conv_kernel(nlayers, bm, k, *refs):
    g_ref, pen_ref = refs[0], refs[1]
    out_ref = refs[-1]
    h = jnp.maximum(g_ref[...], 0.0)
    for i in range(nlayers):
        W = refs[2 + 2 * i][...]
        b = refs[3 + 2 * i][...]
        h = jnp.dot(h, W, preferred_element_type=jnp.float32) + b
        h = jnp.maximum(h, 0.0)
    Cout = h.shape[1]
    h = h + pen_ref[...]
    h3 = h.reshape(bm, k, Cout)
    red = jnp.max(h3, axis=1)
    out_ref[...] = jnp.where(jnp.isneginf(red), 0.0, red)


def _point_conv(layers, g2, pen2, Mp, k, bm=64):
    C = g2.shape[1]
    nl = len(layers)
    Cout = layers[-1][0].shape[1]
    grid = (Mp // bm,)
    in_specs = [
        pl.BlockSpec((bm * k, C), lambda i: (i, 0)),
        pl.BlockSpec((bm * k, 1), lambda i: (i, 0)),
    ]
    args = [g2, pen2]
    for (W, b) in layers:
        in_specs.append(pl.BlockSpec(W.shape, lambda i: (0, 0)))
        in_specs.append(pl.BlockSpec((1, W.shape[1]), lambda i: (0, 0)))
        args.append(W)
        args.append(b.reshape(1, -1))
    out = pl.pallas_call(
        functools.partial(_point_conv_kernel, nl, bm, k),
        grid=grid, in_specs=in_specs,
        out_specs=pl.BlockSpec((bm, Cout), lambda i: (i, 0)),
        out_shape=jax.ShapeDtypeStruct((Mp, Cout), jnp.float32),
    )(*args)
    return out


def _edge_conv(ps, x, ei, n):
    src, dst = ei[0], ei[1]
    (W1, b1) = ps[0]
    C = x.shape[1]
    W1a, W1b = W1[:C], W1[C:]
    h1 = W1.shape[1]
    Wcat = jnp.concatenate([W1a - W1b, W1b], axis=1)
    bcat = jnp.concatenate([b1, jnp.zeros_like(b1)])
    UV = _run_mlp([(Wcat, bcat)], x, 1000, relu_flags=(False,))
    g = UV[dst, :h1] + UV[src, h1:]
    msg = _run_mlp(ps[1:], g, 640, pre_relu=True)
    out = jax.ops.segment_max(msg, dst, num_segments=n)
    return jnp.where(jnp.isneginf(out), 0.0, out)


def _gcu(ps, x, tpl_ei, geo_ei, n):
    xt = _edge_conv(ps['tpl'], x, tpl_ei, n)
    xg = _edge_conv(ps['geo'], x, geo_ei, n)
    return _run_mlp(ps['out'], jnp.concatenate([xt, xg], axis=1), 1000)


def _radius(x, y, r, k=64):
    d2 = jnp.sum((y[:, None, :] - x[None, :, :]) ** 2, axis=-1)
    negd, col = jax.lax.top_k(-d2, k)
    mask = (-negd) < r * r
    return col, mask


def _sa_layer(ps, feat_src, q_src, col, mask, m, Mp, k=64):
    colp = jnp.pad(col, ((0, Mp - m), (0, 0)))
    maskp = jnp.pad(mask, ((0, Mp - m), (0, 0)))
    pen2 = jnp.where(maskp.reshape(-1, 1), 0.0, -jnp.inf).astype(jnp.float32)
    qp = jnp.pad(q_src, ((0, Mp - m), (0, 0)))
    h = feat_src.shape[1]
    g = feat_src[colp.reshape(-1)].reshape(Mp, k, h) - qp[:, None, :]
    g2 = g.reshape(Mp * k, h)
    return _point_conv(ps[1:], g2, pen2, Mp, k)


def _joint_encoder(ps, joints, joints_batch):
    pos = joints
    n = pos.shape[0]
    m1 = int(np.ceil(0.999 * n))
    idx1 = _fps(pos, m1)
    pos1 = pos[idx1]
    col1, mask1 = _radius(pos, pos1, 0.4, 64)
    Mp1 = -(-m1 // 64) * 64
    (W1, b1) = ps['sa1'][0]
    P = _run_mlp([(W1, jnp.zeros_like(b1))], pos, 512, relu_flags=(False,))
    Q = _run_mlp([(W1, -b1)], pos1, 512, relu_flags=(False,))
    x1 = _sa_layer(ps['sa1'], P, Q, col1, mask1, m1, Mp1)[:m1]

    batch1 = joints_batch[idx1]
    m2 = int(np.ceil(0.33 * m1))
    idx2 = _fps(pos1, m2)
    pos2 = pos1[idx2]
    col2, mask2 = _radius(pos1, pos2, 0.6, 64)
    Mp2 = -(-m2 // 64) * 64
    (W1s, b1s) = ps['sa2'][0]
    AP = _run_mlp([(W1s, jnp.zeros_like(b1s))],
                  jnp.concatenate([x1, pos1], axis=1), 512,
                  relu_flags=(False,))
    Wp = W1s[x1.shape[1]:]
    Q2 = _run_mlp([(Wp, -b1s)], pos2, 512, relu_flags=(False,))
    x2 = _sa_layer(ps['sa2'], AP, Q2, col2, mask2, m2, Mp2)[:m2]

    batch2 = batch1[idx2]
    x3in = jnp.concatenate([x2, pos2], axis=1)
    xg = _run_mlp(ps['sa3'], x3in, 512, reduce_rows=True)
    return xg


def _shape_encoder(ps, pos, tpl_ei, geo_ei, batch):
    n = pos.shape[0]
    x1 = _gcu(ps['gcu1'], pos, tpl_ei, geo_ei, n)
    x2 = _gcu(ps['gcu2'], x1, tpl_ei, geo_ei, n)
    x3 = _gcu(ps['gcu3'], x2, tpl_ei, geo_ei, n)
    x4in = jnp.concatenate([x1, x2, x3], axis=1)
    xg = _run_mlp(ps['glb'], x4in, 1000, reduce_rows=True)
    return xg


def kernel(pos, tpl_edge_index, geo_edge_index, batch, joints, joints_batch,
           pairs, pairs_batch, pair_attr, params):
    P = pairs.shape[0]
    jf = _joint_encoder(params['joint'], joints, joints_batch)
    sf = _shape_encoder(params['shape'], pos, tpl_edge_index, geo_edge_index,
                        batch)

    joints_pair = jnp.concatenate(
        [joints[pairs[:, 0]], joints[pairs[:, 1]], pair_attr[:, :-1]], axis=1)
    pf = _run_mlp(params['pair'], joints_pair, 512)

    (Wm1, bm1) = params['mix_mlp'][0]
    counts = jnp.bincount(pairs_batch, length=1)
    sf_r = jnp.repeat(sf, counts, axis=0, total_repeat_length=1)
    jf_r = jnp.repeat(jf, counts, axis=0, total_repeat_length=1)
    b1p = bm1 + (sf_r @ Wm1[:64] + jf_r @ Wm1[64:192])[0]
    Wlin, blin = params['mix_lin']
    layers = [(Wm1[192:], b1p)] + list(params['mix_mlp'][1:]) + [(Wlin, blin)]
    nl = len(layers)
    pre_label = _run_mlp(layers, pf, 512,
                         relu_flags=(True,) * (nl - 1) + (False,))
    gt_label = pair_attr[:, -1:]
    return pre_label, gt_label
